# baseline (speedup 1.0000x reference)
import functools

import jax
import jax.numpy as jnp
from jax import lax
from jax.experimental import pallas as pl
from jax.experimental.pallas import tpu as pltpu
from jax.experimental.pallas import tpu_sc as plsc

NC, NS = 2, 16
NW = NC * NS
ROWS_PER_CHUNK = 8
NBUF = 2
PD = 128
L = 16


def kernel(x, W):
    G0, G1, G2 = x.shape
    B = x.size
    V, D = W.shape
    U = jnp.pad(W, ((0, 0), (0, PD - D)))

    img_elems = G1 * G2
    CHUNK = ROWS_PER_CHUNK * G2
    b_per_w = B // NW
    n_iter = b_per_w // CHUNK
    chunks_per_img = img_elems // CHUNK
    imgs_per_w = G0 // NW

    mesh = plsc.VectorSubcoreMesh(core_axis_name="c", subcore_axis_name="s")

    @functools.partial(
        pl.kernel,
        out_type=jax.ShapeDtypeStruct((G0, G1, G2, D), jnp.float32),
        mesh=mesh,
        scratch_types=[
            pltpu.VMEM((V, PD), jnp.float32),
            pltpu.VMEM((NBUF, ROWS_PER_CHUNK, G2), jnp.int32),
            pltpu.VMEM((NBUF, CHUNK, D), jnp.float32),
            pltpu.SemaphoreType.DMA,
            pltpu.SemaphoreType.DMA,
            pltpu.SemaphoreType.DMA,
            pltpu.SemaphoreType.DMA,
            pltpu.SemaphoreType.DMA,
        ],
        compiler_params=pltpu.CompilerParams(use_tc_tiling_on_sc=True),
    )
    def emb(x_hbm, u_hbm, out_hbm, u_v, idx_v, rows_v, tsem,
            isem0, isem1, osem0, osem1):
        sid = lax.axis_index("s")
        wid = sid * NC + lax.axis_index("c")
        base = wid * b_per_w
        img_base = wid * imgs_per_w
        isems = (isem0, isem1)
        osems = (osem0, osem1)

        pltpu.make_async_copy(u_hbm, u_v, tsem).start()
        pltpu.make_async_copy(u_hbm, u_v, tsem).wait()

        def idx_in(it, b):
            img = img_base + it // chunks_per_img
            r0 = (it % chunks_per_img) * ROWS_PER_CHUNK
            return pltpu.make_async_copy(
                x_hbm.at[img, pl.ds(r0, ROWS_PER_CHUNK)], idx_v.at[b],
                isems[b])

        def rows_out(it, b):
            img = img_base + it // chunks_per_img
            r0 = (it % chunks_per_img) * ROWS_PER_CHUNK
            return [
                pltpu.make_async_copy(
                    rows_v.at[b].at[pl.ds(g * G2, G2)],
                    out_hbm.at[img, r0 + g],
                    osems[b])
                for g in range(ROWS_PER_CHUNK)
            ]

        for b in range(NBUF):
            idx_in(b, b).start()

        def expand(b):
            @plsc.parallel_loop(0, CHUNK // L, unroll=4)
            def grp(g):
                xv = idx_v[b, g // 2, pl.ds((g % 2) * L, L)]
                for ro in range(L):
                    xi = xv[ro]
                    r = g * L + ro
                    vals = [u_v[xi, pl.ds(k * L, L)] for k in range(D // L)]
                    for k in range(D // L):
                        rows_v[b, r, pl.ds(k * L, L)] = vals[k]

        def half(it, b):
            @pl.when(it >= NBUF)
            def _():
                for c in rows_out(it - NBUF, b):
                    c.wait()
            idx_in(it, b).wait()
            expand(b)
            for c in rows_out(it, b):
                c.start()
            @pl.when(it + NBUF < n_iter)
            def _():
                idx_in(it + NBUF, b).start()

        def body(i2, _):
            it = i2 * NBUF
            for b in range(NBUF):
                half(it + b, b)
            return ()

        lax.fori_loop(0, n_iter // NBUF, body, ())
        for b in range(NBUF):
            for c in rows_out(n_iter - NBUF + b, b):
                c.wait()

    return emb(x, U)


# speedup vs baseline: 1.0266x; 1.0266x over previous
import functools

import jax
import jax.numpy as jnp
from jax import lax
from jax.experimental import pallas as pl
from jax.experimental.pallas import tpu as pltpu
from jax.experimental.pallas import tpu_sc as plsc

NC, NS = 2, 16
NW = NC * NS
ROWS_PER_CHUNK = 8
NBUF = 2
PD = 128
L = 16


def kernel(x, W):
    G0, G1, G2 = x.shape
    B = x.size
    V, D = W.shape
    xf = x.reshape(B)
    U = jnp.pad(W, ((0, 0), (0, PD - D)))

    img_elems = G1 * G2
    CHUNK = ROWS_PER_CHUNK * G2
    b_per_w = B // NW
    n_iter = b_per_w // CHUNK
    chunks_per_img = img_elems // CHUNK
    imgs_per_w = G0 // NW

    mesh = plsc.VectorSubcoreMesh(core_axis_name="c", subcore_axis_name="s")

    @functools.partial(
        pl.kernel,
        out_type=jax.ShapeDtypeStruct((G0, G1, G2, D), jnp.float32),
        mesh=mesh,
        scratch_types=[
            pltpu.VMEM((V, PD), jnp.float32),
            pltpu.VMEM((NBUF, CHUNK), jnp.int32),
            pltpu.VMEM((NBUF, CHUNK, D), jnp.float32),
            pltpu.SemaphoreType.DMA,
            pltpu.SemaphoreType.DMA,
            pltpu.SemaphoreType.DMA,
            pltpu.SemaphoreType.DMA,
            pltpu.SemaphoreType.DMA,
        ],
        compiler_params=pltpu.CompilerParams(use_tc_tiling_on_sc=True),
    )
    def emb(x_hbm, u_hbm, out_hbm, u_v, idx_v, rows_v, tsem,
            isem0, isem1, osem0, osem1):
        sid = lax.axis_index("s")
        wid = sid * NC + lax.axis_index("c")
        base = wid * b_per_w
        img_base = wid * imgs_per_w
        isems = (isem0, isem1)
        osems = (osem0, osem1)

        pltpu.make_async_copy(u_hbm, u_v, tsem).start()
        pltpu.make_async_copy(u_hbm, u_v, tsem).wait()

        def idx_in(it, b):
            off = pl.multiple_of(base + it * CHUNK, CHUNK)
            return pltpu.make_async_copy(
                x_hbm.at[pl.ds(off, CHUNK)], idx_v.at[b], isems[b])

        def rows_out(it, b):
            img = img_base + it // chunks_per_img
            r0 = (it % chunks_per_img) * ROWS_PER_CHUNK
            return [
                pltpu.make_async_copy(
                    rows_v.at[b].at[pl.ds(g * G2, G2)],
                    out_hbm.at[img, r0 + g],
                    osems[b])
                for g in range(ROWS_PER_CHUNK)
            ]

        for b in range(NBUF):
            idx_in(b, b).start()

        def expand(b):
            @plsc.parallel_loop(0, CHUNK // L, unroll=4)
            def grp(g):
                xv = idx_v[b, pl.ds(g * L, L)]
                for ro in range(L):
                    xi = xv[ro]
                    r = g * L + ro
                    vals = [u_v[xi, pl.ds(k * L, L)] for k in range(D // L)]
                    for k in range(D // L):
                        rows_v[b, r, pl.ds(k * L, L)] = vals[k]

        def half(it, b):
            @pl.when(it >= NBUF)
            def _():
                for c in rows_out(it - NBUF, b):
                    c.wait()
            idx_in(it, b).wait()
            expand(b)
            for c in rows_out(it, b):
                c.start()
            @pl.when(it + NBUF < n_iter)
            def _():
                idx_in(it + NBUF, b).start()

        def body(i2, _):
            it = i2 * NBUF
            for b in range(NBUF):
                half(it + b, b)
            return ()

        lax.fori_loop(0, n_iter // NBUF, body, ())
        for b in range(NBUF):
            for c in rows_out(n_iter - NBUF + b, b):
                c.wait()

    return emb(xf, U)
